# Initial kernel scaffold; baseline (speedup 1.0000x reference)
import jax
import jax.numpy as jnp
from jax.experimental import pallas as pl


def kernel(x, alpha, rev_w, rev_b, se_w, se_b, sp_w, sp_b, st_w, st_b, te_w, te_b, tp_w, tp_b, tt_w, tt_b):
    raise NotImplementedError("write your pallas kernel here")


# speedup vs baseline: 20.8123x; 20.8123x over previous
import jax
import jax.numpy as jnp
from jax.experimental import pallas as pl
from jax.experimental.pallas import tpu as pltpu

_EPS = 1e-5


def _model_kernel(x_ref, alpha_ref, rev_w_ref, rev_b_ref,
                  se_w_ref, se_b_ref, sp_w_ref, sp_b_ref, st_w_ref, st_b_ref,
                  te_w_ref, te_b_ref, tp_w_ref, tp_b_ref, tt_w_ref, tt_b_ref,
                  out_ref):
    xb = x_ref[0]
    L, C = xb.shape

    mean = jnp.mean(xb, axis=0, keepdims=True)
    var = jnp.mean((xb - mean) ** 2, axis=0, keepdims=True)
    stdev = jnp.sqrt(var + _EPS)
    xn = (xb - mean) / stdev * rev_w_ref[:] + rev_b_ref[:]

    a = jax.nn.sigmoid(alpha_ref[:])
    row_is0 = jax.lax.broadcasted_iota(jnp.int32, (L, 1), 0) == 0
    y = jnp.where(row_is0, xn, (1.0 - a) * xn)
    p = jnp.broadcast_to(a, (1, C))
    d = 1
    while d < L:
        shifted = jnp.concatenate(
            [jnp.zeros((d, C), xb.dtype), y[:L - d]], axis=0)
        y = y + p * shifted
        p = p * p
        d *= 2
    trend = y
    seasonal = xn - trend

    dn = (((0,), (0,)), ((), ()))
    S = jax.lax.dot_general(seasonal, st_w_ref[:], dn,
                            preferred_element_type=jnp.float32)
    T = jax.lax.dot_general(trend, tt_w_ref[:], dn,
                            preferred_element_type=jnp.float32)

    M_s = jnp.dot(se_w_ref[:], sp_w_ref[:], preferred_element_type=jnp.float32)
    M_t = jnp.dot(te_w_ref[:], tp_w_ref[:], preferred_element_type=jnp.float32)

    out = jax.lax.dot_general(S, M_s, dn, preferred_element_type=jnp.float32) \
        + jax.lax.dot_general(T, M_t, dn, preferred_element_type=jnp.float32)

    bs = jnp.dot(se_b_ref[:], sp_w_ref[:],
                 preferred_element_type=jnp.float32) + sp_b_ref[:]
    bt = jnp.dot(te_b_ref[:], tp_w_ref[:],
                 preferred_element_type=jnp.float32) + tp_b_ref[:]
    col_s = jnp.sum(st_w_ref[:], axis=0, keepdims=True)
    col_t = jnp.sum(tt_w_ref[:], axis=0, keepdims=True)
    U = jnp.concatenate([col_s, col_t, st_b_ref[:] + tt_b_ref[:]], axis=0)
    V = jnp.concatenate([bs, bt, jnp.ones((1, C), jnp.float32)], axis=0)
    out = out + jax.lax.dot_general(U, V, dn,
                                    preferred_element_type=jnp.float32)

    out = (out - rev_b_ref[:]) / (rev_w_ref[:] + _EPS)
    out = out * stdev + mean
    out_ref[0] = out


def kernel(x, alpha, rev_w, rev_b, se_w, se_b, sp_w, sp_b, st_w, st_b,
           te_w, te_b, tp_w, tp_b, tt_w, tt_b, interpret=False):
    B, L, C = x.shape
    H = st_w.shape[1]
    D = se_w.shape[1]

    vec = lambda v: v.reshape(1, -1)
    full = lambda s: pl.BlockSpec(s, lambda b: (0,) * len(s))

    return pl.pallas_call(
        _model_kernel,
        grid=(B,),
        in_specs=[
            pl.BlockSpec((1, L, C), lambda b: (b, 0, 0)),
            full((1, C)), full((1, C)), full((1, C)),
            full((C, D)), full((1, D)), full((D, C)), full((1, C)),
            full((L, H)), full((1, H)),
            full((C, D)), full((1, D)), full((D, C)), full((1, C)),
            full((L, H)), full((1, H)),
        ],
        out_specs=pl.BlockSpec((1, H, C), lambda b: (b, 0, 0)),
        out_shape=jax.ShapeDtypeStruct((B, H, C), jnp.float32),
        compiler_params=pltpu.CompilerParams(
            dimension_semantics=("parallel",),
            vmem_limit_bytes=56 * 1024 * 1024,
        ),
        name="starlivts_fused",
        interpret=interpret,
    )(x, vec(alpha), vec(rev_w), vec(rev_b),
      se_w, vec(se_b), sp_w, vec(sp_b), st_w, vec(st_b),
      te_w, vec(te_b), tp_w, vec(tp_b), tt_w, vec(tt_b))


# speedup vs baseline: 25.1253x; 1.2072x over previous
import jax
import jax.numpy as jnp
from jax.experimental import pallas as pl
from jax.experimental.pallas import tpu as pltpu

_EPS = 1e-5


def _model_kernel(x_ref, alpha_ref, rev_w_ref, rev_b_ref,
                  se_w_ref, se_b_ref, sp_w_ref, sp_b_ref, st_w_ref, st_b_ref,
                  te_w_ref, te_b_ref, tp_w_ref, tp_b_ref, tt_w_ref, tt_b_ref,
                  out_ref):
    xb = x_ref[0]
    L, C2 = xb.shape
    C = C2 // 2
    two = lambda v: jnp.concatenate([v, v], axis=1)

    mean = jnp.mean(xb, axis=0, keepdims=True)
    var = jnp.mean((xb - mean) ** 2, axis=0, keepdims=True)
    stdev = jnp.sqrt(var + _EPS)
    rev_w = two(rev_w_ref[:])
    rev_b = two(rev_b_ref[:])
    xn = (xb - mean) / stdev * rev_w + rev_b

    a = two(jax.nn.sigmoid(alpha_ref[:]))
    row_is0 = jax.lax.broadcasted_iota(jnp.int32, (L, 1), 0) == 0
    y = jnp.where(row_is0, xn, (1.0 - a) * xn)
    p = a
    d = 1
    while d < L:
        shifted = jnp.concatenate(
            [jnp.zeros((d, C2), xb.dtype), y[:L - d]], axis=0)
        y = y + p * shifted
        p = p * p
        d *= 2
    trend = y
    seasonal = xn - trend

    dn = (((0,), (0,)), ((), ()))
    S = jax.lax.dot_general(seasonal.astype(jnp.bfloat16), st_w_ref[:], dn,
                            preferred_element_type=jnp.float32)
    T = jax.lax.dot_general(trend.astype(jnp.bfloat16), tt_w_ref[:], dn,
                            preferred_element_type=jnp.float32)

    M_s = jnp.dot(se_w_ref[:], sp_w_ref[:], preferred_element_type=jnp.float32)
    M_t = jnp.dot(te_w_ref[:], tp_w_ref[:], preferred_element_type=jnp.float32)
    z = jnp.zeros((C, C), jnp.float32)
    M2_s = jnp.concatenate([jnp.concatenate([M_s, z], axis=1),
                            jnp.concatenate([z, M_s], axis=1)], axis=0)
    M2_t = jnp.concatenate([jnp.concatenate([M_t, z], axis=1),
                            jnp.concatenate([z, M_t], axis=1)], axis=0)

    out = jax.lax.dot_general(S, M2_s, dn, preferred_element_type=jnp.float32) \
        + jax.lax.dot_general(T, M2_t, dn, preferred_element_type=jnp.float32)

    bs = jnp.dot(se_b_ref[:], sp_w_ref[:],
                 preferred_element_type=jnp.float32) + sp_b_ref[:]
    bt = jnp.dot(te_b_ref[:], tp_w_ref[:],
                 preferred_element_type=jnp.float32) + tp_b_ref[:]
    ones_l = jnp.ones((1, L), jnp.bfloat16)
    dn_lk = (((1,), (0,)), ((), ()))
    col_s = jax.lax.dot_general(ones_l, st_w_ref[:], dn_lk,
                                preferred_element_type=jnp.float32)
    col_t = jax.lax.dot_general(ones_l, tt_w_ref[:], dn_lk,
                                preferred_element_type=jnp.float32)
    U = jnp.concatenate([col_s, col_t, st_b_ref[:] + tt_b_ref[:]], axis=0)
    V = jnp.concatenate([two(bs), two(bt), jnp.ones((1, C2), jnp.float32)],
                        axis=0)
    out = out + jax.lax.dot_general(U, V, dn,
                                    preferred_element_type=jnp.float32)

    out = (out - rev_b) / (rev_w + _EPS)
    out = out * stdev + mean
    out_ref[0] = out


def kernel(x, alpha, rev_w, rev_b, se_w, se_b, sp_w, sp_b, st_w, st_b,
           te_w, te_b, tp_w, tp_b, tt_w, tt_b, interpret=False):
    B, L, C = x.shape
    H = st_w.shape[1]
    D = se_w.shape[1]
    G = B // 2

    xp = x.reshape(G, 2, L, C).transpose(0, 2, 1, 3).reshape(G, L, 2 * C)
    st_bf = st_w.astype(jnp.bfloat16)
    tt_bf = tt_w.astype(jnp.bfloat16)

    vec = lambda v: v.reshape(1, -1)
    full = lambda s: pl.BlockSpec(s, lambda b: (0,) * len(s))

    out2 = pl.pallas_call(
        _model_kernel,
        grid=(G,),
        in_specs=[
            pl.BlockSpec((1, L, 2 * C), lambda b: (b, 0, 0)),
            full((1, C)), full((1, C)), full((1, C)),
            full((C, D)), full((1, D)), full((D, C)), full((1, C)),
            full((L, H)), full((1, H)),
            full((C, D)), full((1, D)), full((D, C)), full((1, C)),
            full((L, H)), full((1, H)),
        ],
        out_specs=pl.BlockSpec((1, H, 2 * C), lambda b: (b, 0, 0)),
        out_shape=jax.ShapeDtypeStruct((G, H, 2 * C), jnp.float32),
        compiler_params=pltpu.CompilerParams(
            dimension_semantics=("parallel",),
            vmem_limit_bytes=56 * 1024 * 1024,
        ),
        name="starlivts_fused",
        interpret=interpret,
    )(xp, vec(alpha), vec(rev_w), vec(rev_b),
      se_w, vec(se_b), sp_w, vec(sp_b), st_bf, vec(st_b),
      te_w, vec(te_b), tp_w, vec(tp_b), tt_bf, vec(tt_b))

    return out2.reshape(G, H, 2, C).transpose(0, 2, 1, 3).reshape(B, H, C)


# speedup vs baseline: 26.1960x; 1.0426x over previous
import jax
import jax.numpy as jnp
from jax.experimental import pallas as pl
from jax.experimental.pallas import tpu as pltpu

_EPS = 1e-5


def _model_kernel(x_ref, alpha_ref, rev_w_ref, rev_b_ref,
                  se_w_ref, se_b_ref, sp_w_ref, sp_b_ref, st_w_ref, st_b_ref,
                  te_w_ref, te_b_ref, tp_w_ref, tp_b_ref, tt_w_ref, tt_b_ref,
                  out_ref, m2s_ref, m2t_ref, bias_ref, y_ref):
    xb = x_ref[0]
    L, C2 = xb.shape
    C = C2 // 2
    H = st_w_ref.shape[1]
    G = pl.num_programs(0)
    i = pl.program_id(0)
    two = lambda v: jnp.concatenate([v, v], axis=1)

    @pl.when((i == 0) | (i == G // 2))
    def _():
        M_s = jnp.dot(se_w_ref[:], sp_w_ref[:],
                      preferred_element_type=jnp.float32)
        M_t = jnp.dot(te_w_ref[:], tp_w_ref[:],
                      preferred_element_type=jnp.float32)
        z = jnp.zeros((C, C), jnp.float32)
        m2s_ref[:] = jnp.concatenate([jnp.concatenate([M_s, z], axis=1),
                                      jnp.concatenate([z, M_s], axis=1)],
                                     axis=0)
        m2t_ref[:] = jnp.concatenate([jnp.concatenate([M_t, z], axis=1),
                                      jnp.concatenate([z, M_t], axis=1)],
                                     axis=0)
        bs = jnp.dot(se_b_ref[:], sp_w_ref[:],
                     preferred_element_type=jnp.float32) + sp_b_ref[:]
        bt = jnp.dot(te_b_ref[:], tp_w_ref[:],
                     preferred_element_type=jnp.float32) + tp_b_ref[:]
        ones_l = jnp.ones((1, L), jnp.bfloat16)
        dn_lk = (((1,), (0,)), ((), ()))
        col_s = jax.lax.dot_general(ones_l, st_w_ref[:], dn_lk,
                                    preferred_element_type=jnp.float32)
        col_t = jax.lax.dot_general(ones_l, tt_w_ref[:], dn_lk,
                                    preferred_element_type=jnp.float32)
        U = jnp.concatenate([col_s, col_t, st_b_ref[:] + tt_b_ref[:]], axis=0)
        V = jnp.concatenate([two(bs), two(bt), jnp.ones((1, C2), jnp.float32)],
                            axis=0)
        bias_ref[:] = jax.lax.dot_general(
            U, V, (((0,), (0,)), ((), ())),
            preferred_element_type=jnp.float32)

    mean = jnp.mean(xb, axis=0, keepdims=True)
    var = jnp.mean((xb - mean) ** 2, axis=0, keepdims=True)
    stdev = jnp.sqrt(var + _EPS)
    rev_w = two(rev_w_ref[:])
    rev_b = two(rev_b_ref[:])
    xn = (xb - mean) / stdev * rev_w + rev_b

    a = two(jax.nn.sigmoid(alpha_ref[:]))
    row_is0 = jax.lax.broadcasted_iota(jnp.int32, (L, 1), 0) == 0
    y = jnp.where(row_is0, xn, (1.0 - a) * xn)
    p = a
    d = 1
    while d < 64:
        shifted = jnp.concatenate(
            [jnp.zeros((d, C2), xb.dtype), y[:L - d]], axis=0)
        y = y + p * shifted
        p = p * p
        d *= 2
    y_ref[:] = y

    @pl.when(jnp.max(p) > 1e-10)
    def _():
        yy = y_ref[:]
        pp = p
        dd = 64
        while dd < L:
            sh = jnp.concatenate(
                [jnp.zeros((dd, C2), xb.dtype), yy[:L - dd]], axis=0)
            yy = yy + pp * sh
            pp = pp * pp
            dd *= 2
        y_ref[:] = yy

    trend = y_ref[:]
    seasonal = xn - trend

    dn = (((0,), (0,)), ((), ()))
    S = jax.lax.dot_general(st_w_ref[:], seasonal.astype(jnp.bfloat16), dn,
                            preferred_element_type=jnp.float32)
    T = jax.lax.dot_general(tt_w_ref[:], trend.astype(jnp.bfloat16), dn,
                            preferred_element_type=jnp.float32)

    dn_std = (((1,), (0,)), ((), ()))
    out = jax.lax.dot_general(S, m2s_ref[:], dn_std,
                              preferred_element_type=jnp.float32) \
        + jax.lax.dot_general(T, m2t_ref[:], dn_std,
                              preferred_element_type=jnp.float32) \
        + bias_ref[:]

    out = (out - rev_b) / (rev_w + _EPS)
    out = out * stdev + mean
    out_ref[0, 0] = out[:, :C]
    out_ref[0, 1] = out[:, C:]


def kernel(x, alpha, rev_w, rev_b, se_w, se_b, sp_w, sp_b, st_w, st_b,
           te_w, te_b, tp_w, tp_b, tt_w, tt_b, interpret=False):
    B, L, C = x.shape
    H = st_w.shape[1]
    D = se_w.shape[1]
    G = B // 2

    xp = x.reshape(G, 2, L, C).transpose(0, 2, 1, 3).reshape(G, L, 2 * C)
    st_bf = st_w.astype(jnp.bfloat16)
    tt_bf = tt_w.astype(jnp.bfloat16)

    vec = lambda v: v.reshape(1, -1)
    full = lambda s: pl.BlockSpec(s, lambda b: (0,) * len(s))

    out2 = pl.pallas_call(
        _model_kernel,
        grid=(G,),
        in_specs=[
            pl.BlockSpec((1, L, 2 * C), lambda b: (b, 0, 0)),
            full((1, C)), full((1, C)), full((1, C)),
            full((C, D)), full((1, D)), full((D, C)), full((1, C)),
            full((L, H)), full((1, H)),
            full((C, D)), full((1, D)), full((D, C)), full((1, C)),
            full((L, H)), full((1, H)),
        ],
        out_specs=pl.BlockSpec((1, 2, H, C), lambda b: (b, 0, 0, 0)),
        out_shape=jax.ShapeDtypeStruct((G, 2, H, C), jnp.float32),
        scratch_shapes=[
            pltpu.VMEM((2 * C, 2 * C), jnp.float32),
            pltpu.VMEM((2 * C, 2 * C), jnp.float32),
            pltpu.VMEM((H, 2 * C), jnp.float32),
            pltpu.VMEM((L, 2 * C), jnp.float32),
        ],
        compiler_params=pltpu.CompilerParams(
            dimension_semantics=("parallel",),
            vmem_limit_bytes=56 * 1024 * 1024,
        ),
        name="starlivts_fused",
        interpret=interpret,
    )(xp, vec(alpha), vec(rev_w), vec(rev_b),
      se_w, vec(se_b), sp_w, vec(sp_b), st_bf, vec(st_b),
      te_w, vec(te_b), tp_w, vec(tp_b), tt_bf, vec(tt_b))

    return out2.reshape(B, H, C)


# speedup vs baseline: 28.5353x; 1.0893x over previous
import jax
import jax.numpy as jnp
from jax.experimental import pallas as pl
from jax.experimental.pallas import tpu as pltpu

_EPS = 1e-5
_PAIRS = 2


def _model_kernel(x_ref, alpha_ref, rev_w_ref, rev_b_ref,
                  se_w_ref, se_b_ref, sp_w_ref, sp_b_ref, st_w_ref, st_b_ref,
                  te_w_ref, te_b_ref, tp_w_ref, tp_b_ref, tt_w_ref, tt_b_ref,
                  out_ref, m2s_ref, m2t_ref, bias_ref, colt_ref, y_ref):
    xb = x_ref[0]
    P, L, C2 = xb.shape
    C = C2 // 2
    G = pl.num_programs(0)
    i = pl.program_id(0)
    two = lambda v: jnp.concatenate([v, v], axis=1)
    rev_w = two(rev_w_ref[:])
    rev_b = two(rev_b_ref[:])

    @pl.when((i == 0) | (i == G // 2))
    def _():
        M_s = jnp.dot(se_w_ref[:], sp_w_ref[:],
                      preferred_element_type=jnp.float32)
        M_t = jnp.dot(te_w_ref[:], tp_w_ref[:],
                      preferred_element_type=jnp.float32)
        z = jnp.zeros((C, C), jnp.float32)
        m2s_ref[:] = jnp.concatenate([jnp.concatenate([M_s, z], axis=1),
                                      jnp.concatenate([z, M_s], axis=1)],
                                     axis=0)
        m2t_ref[:] = jnp.concatenate([jnp.concatenate([M_t, z], axis=1),
                                      jnp.concatenate([z, M_t], axis=1)],
                                     axis=0)
        bs = jnp.dot(se_b_ref[:], sp_w_ref[:],
                     preferred_element_type=jnp.float32) + sp_b_ref[:]
        bt = jnp.dot(te_b_ref[:], tp_w_ref[:],
                     preferred_element_type=jnp.float32) + tp_b_ref[:]
        ones_l = jnp.ones((1, L), jnp.bfloat16)
        dn_lk = (((1,), (0,)), ((), ()))
        col_s = jax.lax.dot_general(ones_l, st_w_ref[:], dn_lk,
                                    preferred_element_type=jnp.float32)
        col_t = jax.lax.dot_general(ones_l, tt_w_ref[:], dn_lk,
                                    preferred_element_type=jnp.float32)
        colt_ref[:] = col_t
        U = jnp.concatenate([col_s, col_t, st_b_ref[:] + tt_b_ref[:]], axis=0)
        V = jnp.concatenate([two(bs), two(bt), jnp.ones((1, C2), jnp.float32)],
                            axis=0)
        bias_ref[:] = jax.lax.dot_general(
            U, V, (((0,), (0,)), ((), ())),
            preferred_element_type=jnp.float32)

    mean = jnp.mean(xb, axis=1, keepdims=True)
    var = jnp.mean((xb - mean) ** 2, axis=1, keepdims=True)
    stdev = jnp.sqrt(var + _EPS)
    s = rev_w / stdev
    o = rev_b - mean * s

    a = two(jax.nn.sigmoid(alpha_ref[:]))
    row_is0 = jax.lax.broadcasted_iota(jnp.int32, (1, L, 1), 1) == 0
    y = jnp.where(row_is0, xb, (1.0 - a) * xb)
    p = a
    d = 1
    while d < 64:
        shifted = jnp.concatenate(
            [jnp.zeros((P, d, C2), xb.dtype), y[:, :L - d]], axis=1)
        y = y + p * shifted
        p = p * p
        d *= 2
    y_ref[:] = y

    @pl.when(jnp.max(p) > 1e-10)
    def _():
        yy = y_ref[:]
        pp = p
        dd = 64
        while dd < L:
            sh = jnp.concatenate(
                [jnp.zeros((P, dd, C2), xb.dtype), yy[:, :L - dd]], axis=1)
            yy = yy + pp * sh
            pp = pp * pp
            dd *= 2
        y_ref[:] = yy

    tr = y_ref[:]
    diff = xb - tr

    dn0 = (((0,), (0,)), ((), ()))
    dn_std = (((1,), (0,)), ((), ()))
    inv_rw = 1.0 / (rev_w + _EPS)
    for j in range(P):
        sj = s[j]
        Sr = jax.lax.dot_general(st_w_ref[:], diff[j].astype(jnp.bfloat16),
                                 dn0, preferred_element_type=jnp.float32)
        Tr = jax.lax.dot_general(tt_w_ref[:], tr[j].astype(jnp.bfloat16),
                                 dn0, preferred_element_type=jnp.float32)
        oM = jax.lax.dot_general(o[j], m2t_ref[:], dn_std,
                                 preferred_element_type=jnp.float32)
        outp = jax.lax.dot_general(Sr * sj, m2s_ref[:], dn_std,
                                   preferred_element_type=jnp.float32) \
            + jax.lax.dot_general(Tr * sj, m2t_ref[:], dn_std,
                                  preferred_element_type=jnp.float32) \
            + jax.lax.dot_general(colt_ref[:], oM, dn0,
                                  preferred_element_type=jnp.float32) \
            + bias_ref[:]
        q = inv_rw * stdev[j]
        r = mean[j] - rev_b * q
        out = outp * q + r
        out_ref[0, j, 0] = out[:, :C]
        out_ref[0, j, 1] = out[:, C:]


def kernel(x, alpha, rev_w, rev_b, se_w, se_b, sp_w, sp_b, st_w, st_b,
           te_w, te_b, tp_w, tp_b, tt_w, tt_b, interpret=False):
    B, L, C = x.shape
    H = st_w.shape[1]
    D = se_w.shape[1]
    P = _PAIRS
    G = B // (2 * P)

    xp = x.reshape(G, P, 2, L, C).transpose(0, 1, 3, 2, 4).reshape(
        G, P, L, 2 * C)
    st_bf = st_w.astype(jnp.bfloat16)
    tt_bf = tt_w.astype(jnp.bfloat16)

    vec = lambda v: v.reshape(1, -1)
    full = lambda s: pl.BlockSpec(s, lambda b: (0,) * len(s))

    out2 = pl.pallas_call(
        _model_kernel,
        grid=(G,),
        in_specs=[
            pl.BlockSpec((1, P, L, 2 * C), lambda b: (b, 0, 0, 0)),
            full((1, C)), full((1, C)), full((1, C)),
            full((C, D)), full((1, D)), full((D, C)), full((1, C)),
            full((L, H)), full((1, H)),
            full((C, D)), full((1, D)), full((D, C)), full((1, C)),
            full((L, H)), full((1, H)),
        ],
        out_specs=pl.BlockSpec((1, P, 2, H, C), lambda b: (b, 0, 0, 0, 0)),
        out_shape=jax.ShapeDtypeStruct((G, P, 2, H, C), jnp.float32),
        scratch_shapes=[
            pltpu.VMEM((2 * C, 2 * C), jnp.float32),
            pltpu.VMEM((2 * C, 2 * C), jnp.float32),
            pltpu.VMEM((H, 2 * C), jnp.float32),
            pltpu.VMEM((1, H), jnp.float32),
            pltpu.VMEM((P, L, 2 * C), jnp.float32),
        ],
        compiler_params=pltpu.CompilerParams(
            dimension_semantics=("parallel",),
            vmem_limit_bytes=56 * 1024 * 1024,
        ),
        name="starlivts_fused",
        interpret=interpret,
    )(xp, vec(alpha), vec(rev_w), vec(rev_b),
      se_w, vec(se_b), sp_w, vec(sp_b), st_bf, vec(st_b),
      te_w, vec(te_b), tp_w, vec(tp_b), tt_bf, vec(tt_b))

    return out2.reshape(B, H, C)
